# baseline (speedup 1.0000x reference)
import functools

import jax
import jax.numpy as jnp
from jax import lax
from jax.experimental import pallas as pl
from jax.experimental.pallas import tpu as pltpu
from jax.experimental.pallas import tpu_sc as plsc

N = 10000
G = 128
D = 128
L = 4
K = 32
CUTOFF = 5.0
EPS = 1e-5

NPAD = 10240
RT = 128
NRT = NPAD // RT
CT = 512
BIG = 1e30
PADB = 1 << 20

E = NPAD * K
NB = 16
EB = NB * K

NW = 32
EPW = E // NW
CH = 128
NCH = EPW // CH


def _nbr_kernel(tlo_ref, thi_ref, brow_ref, posr_ref, post_ref, batch_ref,
                nbr_ref, emask_ref, d2_ref, band):
    pid = pl.program_id(0)
    r0 = pid * RT
    pos_r = posr_ref[...]
    sq_r = jnp.sum(pos_r * pos_r, axis=1, keepdims=True)
    b_r = brow_ref[0]
    row_ids = r0 + lax.broadcasted_iota(jnp.int32, (RT, 1), 0)
    t_lo = tlo_ref[pid]
    t_hi = thi_ref[pid]

    def build(t, carry):
        c0 = t * CT
        pos_c = post_ref[:, pl.ds(c0, CT)]
        sq_c = jnp.sum(pos_c * pos_c, axis=0, keepdims=True)
        b_c = batch_ref[:, pl.ds(c0, CT)]
        d2 = sq_r + sq_c - 2.0 * jnp.dot(pos_r, pos_c,
                                         preferred_element_type=jnp.float32)
        d2 = jnp.maximum(d2, 0.0)
        col_ids = c0 + lax.broadcasted_iota(jnp.int32, (RT, CT), 1)
        valid = (b_r == b_c) & (row_ids != col_ids) & (d2 <= CUTOFF * CUTOFF)
        band[:, pl.ds(c0, CT)] = jnp.where(valid, d2, BIG)
        return carry

    lax.fori_loop(t_lo, t_hi, build, 0)

    prev = jnp.full((RT, 1), -1, jnp.int32)
    idxs = []
    vals = []
    for _ in range(K):
        def sel(t, carry):
            run_v, run_i, pv = carry
            c0 = t * CT
            tile = band[:, pl.ds(c0, CT)]
            col_ids = c0 + lax.broadcasted_iota(jnp.int32, (RT, CT), 1)
            tile = jnp.where(col_ids == pv, BIG, tile)
            band[:, pl.ds(c0, CT)] = tile
            tmin = jnp.min(tile, axis=1, keepdims=True)
            targ = jnp.min(jnp.where(tile == tmin, col_ids, jnp.int32(NPAD)),
                           axis=1, keepdims=True)
            upd = tmin < run_v
            return (jnp.where(upd, tmin, run_v),
                    jnp.where(upd, targ, run_i), pv)

        v0 = jnp.full((RT, 1), BIG, jnp.float32)
        i0 = jnp.zeros((RT, 1), jnp.int32)
        val, idx, _ = lax.fori_loop(t_lo, t_hi, sel, (v0, i0, prev))
        idxs.append(idx)
        vals.append(val)
        prev = idx

    nbr_ref[...] = jnp.concatenate(idxs, axis=1)
    valm = jnp.concatenate(vals, axis=1)
    emask_ref[...] = jnp.where((valm < BIG * 0.5) & (row_ids < N), 1.0, 0.0)
    d2_ref[...] = jnp.minimum(valm, CUTOFF * CUTOFF)


def _build_neighbors(tlo, thi, brow, pos8, posT, batch2d):
    grid_spec = pltpu.PrefetchScalarGridSpec(
        num_scalar_prefetch=2,
        grid=(NRT,),
        in_specs=[
            pl.BlockSpec((1, RT, 1), lambda i, *_: (i, 0, 0)),
            pl.BlockSpec((RT, 8), lambda i, *_: (i, 0)),
            pl.BlockSpec((8, NPAD), lambda i, *_: (0, 0)),
            pl.BlockSpec((1, NPAD), lambda i, *_: (0, 0)),
        ],
        out_specs=[
            pl.BlockSpec((RT, K), lambda i, *_: (i, 0)),
            pl.BlockSpec((RT, K), lambda i, *_: (i, 0)),
            pl.BlockSpec((RT, K), lambda i, *_: (i, 0)),
        ],
        scratch_shapes=[pltpu.VMEM((RT, NPAD), jnp.float32)],
    )
    return pl.pallas_call(
        _nbr_kernel,
        grid_spec=grid_spec,
        out_shape=[
            jax.ShapeDtypeStruct((NPAD, K), jnp.int32),
            jax.ShapeDtypeStruct((NPAD, K), jnp.float32),
            jax.ShapeDtypeStruct((NPAD, K), jnp.float32),
        ],
    )(tlo, thi, brow, pos8, posT, batch2d)


NBUF = 4
NGRP = NCH // NBUF


def _make_sc_gather(ncols):
    mesh = plsc.VectorSubcoreMesh(core_axis_name="c", subcore_axis_name="s")

    @functools.partial(
        pl.kernel,
        mesh=mesh,
        out_type=jax.ShapeDtypeStruct((E, ncols), jnp.float32),
        scratch_types=[
            pltpu.VMEM((EPW,), jnp.int32),
        ] + [pltpu.VMEM((CH, ncols), jnp.float32) for _ in range(NBUF)]
        + [pltpu.SemaphoreType.DMA for _ in range(NBUF)],
    )
    def k(table_hbm, idx_hbm, out_hbm, idx_v, *bufs_sems):
        rows = bufs_sems[:NBUF]
        sems = bufs_sems[NBUF:]
        wid = lax.axis_index("s") * 2 + lax.axis_index("c")
        base = wid * EPW
        pltpu.sync_copy(idx_hbm.at[pl.ds(base, EPW)], idx_v)

        for b in range(NBUF):
            pltpu.async_copy(table_hbm.at[idx_v.at[pl.ds(b * CH, CH)]],
                             rows[b], sems[b])

        def body(g, carry):
            c0 = g * NBUF
            for b in range(NBUF):
                pltpu.make_async_copy(
                    table_hbm.at[idx_v.at[pl.ds(0, CH)]],
                    rows[b], sems[b]).wait()
                pltpu.sync_copy(rows[b],
                                out_hbm.at[pl.ds(base + (c0 + b) * CH, CH)])

            @pl.when(g + 1 < NGRP)
            def _():
                for b in range(NBUF):
                    off = (c0 + NBUF + b) * CH
                    pltpu.async_copy(
                        table_hbm.at[idx_v.at[pl.ds(off, CH)]],
                        rows[b], sems[b])

            return carry

        lax.fori_loop(0, NGRP, body, 0)

    return k


@functools.lru_cache(maxsize=None)
def _get_gather(ncols):
    return _make_sc_gather(ncols)


def _init_kernel(z_ref, emb_ref, wf_ref, bfv_ref, ws_ref, bsv_ref,
                 x_ref, af_ref, as_ref):
    zc = z_ref[...]
    oh = (zc == lax.broadcasted_iota(jnp.int32, (NPAD, D), 1))
    x = jnp.dot(oh.astype(jnp.float32), emb_ref[...],
                preferred_element_type=jnp.float32)
    x_ref[...] = x
    af_ref[...] = jnp.dot(x, wf_ref[...],
                          preferred_element_type=jnp.float32) + bfv_ref[...]
    as_ref[...] = jnp.dot(x, ws_ref[...],
                          preferred_element_type=jnp.float32) + bsv_ref[...]


def _init_call(zcol, embp, wf0, bf0, ws0, bs0):
    return pl.pallas_call(
        _init_kernel,
        out_shape=[jax.ShapeDtypeStruct((NPAD, D), jnp.float32)] * 3,
    )(zcol, embp, wf0, bf0, ws0, bs0)


def _dist_kernel(posg_ref, posd_ref, dist_ref):
    pdst = jnp.broadcast_to(posd_ref[...][:, None, :],
                            (NB, K, D)).reshape(EB, D)
    diff = posg_ref[...] - pdst
    dist_ref[...] = jnp.sqrt(jnp.sum(diff * diff, axis=1, keepdims=True)
                             + 1e-12)


def _dist_call(posg, pos128):
    return pl.pallas_call(
        _dist_kernel,
        grid=(E // EB,),
        in_specs=[
            pl.BlockSpec((EB, D), lambda i: (i, 0)),
            pl.BlockSpec((NB, D), lambda i: (i, 0)),
        ],
        out_specs=pl.BlockSpec((EB, 1), lambda i: (i, 0)),
        out_shape=jax.ShapeDtypeStruct((E, 1), jnp.float32),
    )(posg, pos128)


def _edge_kernel(xg_ref, dist_ref, af_ref, as_ref, em_ref,
                 wfn_ref, wsn_ref, wfd_ref, wsd_ref, aggr_ref):
    xg = xg_ref[...]
    mmf = jnp.dot(xg, wfn_ref[...], preferred_element_type=jnp.float32)
    mms = jnp.dot(xg, wsn_ref[...], preferred_element_type=jnp.float32)
    dist = dist_ref[...]
    afb = jnp.broadcast_to(af_ref[...][:, None, :], (NB, K, D)).reshape(EB, D)
    asb = jnp.broadcast_to(as_ref[...][:, None, :], (NB, K, D)).reshape(EB, D)
    gate = jax.nn.sigmoid(afb + mmf + dist * wfd_ref[...])
    core = jax.nn.softplus(asb + mms + dist * wsd_ref[...])
    msg = gate * core * em_ref[...]
    aggr_ref[...] = jnp.sum(msg.reshape(NB, K, D), axis=1)


def _edge_call(xg, dist_e, af, asv, em_e, wfn, wsn, wfd, wsd):
    return pl.pallas_call(
        _edge_kernel,
        grid=(E // EB,),
        in_specs=[
            pl.BlockSpec((EB, D), lambda i: (i, 0)),
            pl.BlockSpec((EB, 1), lambda i: (i, 0)),
            pl.BlockSpec((NB, D), lambda i: (i, 0)),
            pl.BlockSpec((NB, D), lambda i: (i, 0)),
            pl.BlockSpec((EB, 1), lambda i: (i, 0)),
            pl.BlockSpec((D, D), lambda i: (0, 0)),
            pl.BlockSpec((D, D), lambda i: (0, 0)),
            pl.BlockSpec((1, D), lambda i: (0, 0)),
            pl.BlockSpec((1, D), lambda i: (0, 0)),
        ],
        out_specs=pl.BlockSpec((NB, D), lambda i: (i, 0)),
        out_shape=jax.ShapeDtypeStruct((NPAD, D), jnp.float32),
    )(xg, dist_e, af, asv, em_e, wfn, wsn, wfd, wsd)


def _bn_kernel(aggr_ref, x_ref, g1_ref, b1_ref, g2_ref, b2_ref,
               wfn_ref, bfn_ref, wsn_ref, bsn_ref,
               xn_ref, af_ref, as_ref):
    rmask = (lax.broadcasted_iota(jnp.int32, (NPAD, 1), 0) < N)
    rmf = rmask.astype(jnp.float32)
    a = aggr_ref[...] * rmf
    s1 = jnp.sum(a, axis=0, keepdims=True) / N
    v1 = jnp.sum(a * a, axis=0, keepdims=True) / N - s1 * s1
    out = (a - s1) / jnp.sqrt(v1 + EPS) * g1_ref[...] + b1_ref[...] \
        + x_ref[...]
    om = out * rmf
    s2 = jnp.sum(om, axis=0, keepdims=True) / N
    v2 = jnp.sum(om * om, axis=0, keepdims=True) / N - s2 * s2
    xn = jnp.maximum((out - s2) / jnp.sqrt(v2 + EPS) * g2_ref[...]
                     + b2_ref[...], 0.0) * rmf
    xn_ref[...] = xn
    af_ref[...] = jnp.dot(xn, wfn_ref[...],
                          preferred_element_type=jnp.float32) + bfn_ref[...]
    as_ref[...] = jnp.dot(xn, wsn_ref[...],
                          preferred_element_type=jnp.float32) + bsn_ref[...]


def _bn_call(aggr, x, g1, b1, g2, b2, wfn, bfn, wsn, bsn):
    return pl.pallas_call(
        _bn_kernel,
        out_shape=[jax.ShapeDtypeStruct((NPAD, D), jnp.float32)] * 3,
    )(aggr, x, g1, b1, g2, b2, wfn, bfn, wsn, bsn)


def _head_kernel(x_ref, bcol_ref, w1_ref, b1_ref, w2_ref, b2_ref, out_ref):
    oh = (bcol_ref[...] == lax.broadcasted_iota(jnp.int32, (NPAD, G), 1))
    ohf = oh.astype(jnp.float32)
    seg = lax.dot_general(ohf, x_ref[...], (((0,), (0,)), ((), ())),
                          preferred_element_type=jnp.float32)
    ones = jnp.ones((NPAD, 1), jnp.float32)
    cnt = lax.dot_general(ohf, ones, (((0,), (0,)), ((), ())),
                          preferred_element_type=jnp.float32)
    gfeat = seg / jnp.maximum(cnt, 1.0)
    h = jnp.maximum(jnp.dot(gfeat, w1_ref[...],
                            preferred_element_type=jnp.float32)
                    + b1_ref[...], 0.0)
    out_ref[...] = jnp.sum(h * w2_ref[...], axis=1, keepdims=True) \
        + b2_ref[...]


def _head_call(x, bcol, hW1, hb1, hw2r, hb2):
    return pl.pallas_call(
        _head_kernel,
        out_shape=jax.ShapeDtypeStruct((G, 1), jnp.float32),
    )(x, bcol, hW1, hb1, hw2r, hb2)


def kernel(z, pos, batch, emb, Wf, bf, Ws, bs, bn1_g, bn1_b, bn2_g, bn2_b,
           hW1, hb1, hW2, hb2):
    batchp = jnp.concatenate(
        [batch.astype(jnp.int32), jnp.full((NPAD - N,), PADB, jnp.int32)])
    posp = jnp.concatenate(
        [pos.astype(jnp.float32), jnp.zeros((NPAD - N, 3), jnp.float32)], 0)
    pos8 = jnp.pad(posp, ((0, 0), (0, 5)))
    posT = pos8.T
    zcol = jnp.concatenate(
        [z.astype(jnp.int32), jnp.zeros((NPAD - N,), jnp.int32)]).reshape(
            NPAD, 1)
    embp = jnp.pad(emb.astype(jnp.float32), ((0, D - emb.shape[0]), (0, 0)))

    bt = batchp.reshape(NRT, RT)
    lo = jnp.searchsorted(batchp, bt.min(axis=1), side="left")
    hi = jnp.searchsorted(batchp, bt.max(axis=1), side="right")
    tlo = (lo // CT).astype(jnp.int32)
    thi = ((hi + CT - 1) // CT).astype(jnp.int32)
    brow = batchp.reshape(NRT, RT, 1)
    batch2d = batchp.reshape(1, NPAD)
    bcol = batchp.reshape(NPAD, 1)

    wf_self = [Wf[l, :D, :] for l in range(L)]
    wf_nbr = [Wf[l, D:2 * D, :] for l in range(L)]
    wf_d = [Wf[l, 2 * D:, :] for l in range(L)]
    ws_self = [Ws[l, :D, :] for l in range(L)]
    ws_nbr = [Ws[l, D:2 * D, :] for l in range(L)]
    ws_d = [Ws[l, 2 * D:, :] for l in range(L)]
    bfv = [bf[l].reshape(1, D) for l in range(L)]
    bsv = [bs[l].reshape(1, D) for l in range(L)]
    g1 = [bn1_g[l].reshape(1, D) for l in range(L)]
    b1 = [bn1_b[l].reshape(1, D) for l in range(L)]
    g2 = [bn2_g[l].reshape(1, D) for l in range(L)]
    b2 = [bn2_b[l].reshape(1, D) for l in range(L)]

    nbr, emask, _ = _build_neighbors(tlo, thi, brow, pos8, posT, batch2d)
    idx_flat = nbr.reshape(E)
    em_e = emask.reshape(E, 1)

    pos128 = jnp.pad(posp, ((0, 0), (0, D - 3)))
    posg = _get_gather(D)(pos128, idx_flat)
    dist_e = _dist_call(posg, pos128)
    x, af, asv = _init_call(zcol, embp, wf_self[0], bfv[0], ws_self[0],
                            bsv[0])
    for l in range(L):
        ln = (l + 1) % L
        xg = _get_gather(D)(x, idx_flat)
        aggr = _edge_call(xg, dist_e, af, asv, em_e,
                          wf_nbr[l], ws_nbr[l], wf_d[l], ws_d[l])
        x, af, asv = _bn_call(aggr, x, g1[l], b1[l], g2[l], b2[l],
                              wf_self[ln], bfv[ln], ws_self[ln], bsv[ln])

    out = _head_call(x, bcol, hW1, hb1.reshape(1, D), hW2.reshape(1, D),
                     hb2.reshape(1, 1))
    return out.reshape(-1)


# speedup vs baseline: 6.1366x; 6.1366x over previous
import functools

import jax
import jax.numpy as jnp
from jax import lax
from jax.experimental import pallas as pl
from jax.experimental.pallas import tpu as pltpu
from jax.experimental.pallas import tpu_sc as plsc

N = 10000
G = 128
D = 128
L = 4
K = 32
CUTOFF = 5.0
EPS = 1e-5

NPAD = 10240
RT = 128
NRT = NPAD // RT
CT = 512
BIG = 1e30
PADB = 1 << 20

E = NPAD * K
NB = 16
EB = NB * K

NW = 32
EPW = E // NW
CH = 32
NCH = EPW // CH


def _nbr_kernel(tlo_ref, thi_ref, brow_ref, posr_ref, post_ref, batch_ref,
                nbr_ref, emask_ref, d2_ref, band):
    pid = pl.program_id(0)
    r0 = pid * RT
    pos_r = posr_ref[...]
    sq_r = jnp.sum(pos_r * pos_r, axis=1, keepdims=True)
    b_r = brow_ref[0]
    row_ids = r0 + lax.broadcasted_iota(jnp.int32, (RT, 1), 0)
    t_lo = tlo_ref[pid]
    t_hi = thi_ref[pid]

    def build(t, carry):
        c0 = t * CT
        pos_c = post_ref[:, pl.ds(c0, CT)]
        sq_c = jnp.sum(pos_c * pos_c, axis=0, keepdims=True)
        b_c = batch_ref[:, pl.ds(c0, CT)]
        d2 = sq_r + sq_c - 2.0 * jnp.dot(pos_r, pos_c,
                                         preferred_element_type=jnp.float32)
        d2 = jnp.maximum(d2, 0.0)
        col_ids = c0 + lax.broadcasted_iota(jnp.int32, (RT, CT), 1)
        valid = (b_r == b_c) & (row_ids != col_ids) & (d2 <= CUTOFF * CUTOFF)
        band[:, pl.ds(c0, CT)] = jnp.where(valid, d2, BIG)
        return carry

    lax.fori_loop(t_lo, t_hi, build, 0)

    prev = jnp.full((RT, 1), -1, jnp.int32)
    idxs = []
    vals = []
    for _ in range(K):
        def sel(t, carry):
            run_v, run_i, pv = carry
            c0 = t * CT
            tile = band[:, pl.ds(c0, CT)]
            col_ids = c0 + lax.broadcasted_iota(jnp.int32, (RT, CT), 1)
            tile = jnp.where(col_ids == pv, BIG, tile)
            band[:, pl.ds(c0, CT)] = tile
            tmin = jnp.min(tile, axis=1, keepdims=True)
            targ = jnp.min(jnp.where(tile == tmin, col_ids, jnp.int32(NPAD)),
                           axis=1, keepdims=True)
            upd = tmin < run_v
            return (jnp.where(upd, tmin, run_v),
                    jnp.where(upd, targ, run_i), pv)

        v0 = jnp.full((RT, 1), BIG, jnp.float32)
        i0 = jnp.zeros((RT, 1), jnp.int32)
        val, idx, _ = lax.fori_loop(t_lo, t_hi, sel, (v0, i0, prev))
        idxs.append(idx)
        vals.append(val)
        prev = idx

    nbr_ref[...] = jnp.concatenate(idxs, axis=1)
    valm = jnp.concatenate(vals, axis=1)
    emask_ref[...] = jnp.where((valm < BIG * 0.5) & (row_ids < N), 1.0, 0.0)
    d2_ref[...] = jnp.minimum(valm, CUTOFF * CUTOFF)


def _build_neighbors(tlo, thi, brow, pos8, posT, batch2d):
    grid_spec = pltpu.PrefetchScalarGridSpec(
        num_scalar_prefetch=2,
        grid=(NRT,),
        in_specs=[
            pl.BlockSpec((1, RT, 1), lambda i, *_: (i, 0, 0)),
            pl.BlockSpec((RT, 8), lambda i, *_: (i, 0)),
            pl.BlockSpec((8, NPAD), lambda i, *_: (0, 0)),
            pl.BlockSpec((1, NPAD), lambda i, *_: (0, 0)),
        ],
        out_specs=[
            pl.BlockSpec((RT, K), lambda i, *_: (i, 0)),
            pl.BlockSpec((RT, K), lambda i, *_: (i, 0)),
            pl.BlockSpec((RT, K), lambda i, *_: (i, 0)),
        ],
        scratch_shapes=[pltpu.VMEM((RT, NPAD), jnp.float32)],
    )
    return pl.pallas_call(
        _nbr_kernel,
        grid_spec=grid_spec,
        out_shape=[
            jax.ShapeDtypeStruct((NPAD, K), jnp.int32),
            jax.ShapeDtypeStruct((NPAD, K), jnp.float32),
            jax.ShapeDtypeStruct((NPAD, K), jnp.float32),
        ],
    )(tlo, thi, brow, pos8, posT, batch2d)


NBUF = 4
NGRP = NCH // NBUF


def _make_sc_gather(ncols):
    mesh = plsc.VectorSubcoreMesh(core_axis_name="c", subcore_axis_name="s")

    @functools.partial(
        pl.kernel,
        mesh=mesh,
        out_type=jax.ShapeDtypeStruct((E, ncols), jnp.float32),
        scratch_types=[
            pltpu.VMEM((EPW,), jnp.int32),
            pltpu.VMEM_SHARED((NPAD, ncols), jnp.float32),
        ] + [pltpu.VMEM((CH, ncols), jnp.float32) for _ in range(NBUF)]
        + [pltpu.SemaphoreType.DMA for _ in range(2 * NBUF)],
    )
    def k(table_hbm, idx_hbm, out_hbm, idx_v, tab_s, *bufs_sems):
        rows = bufs_sems[:NBUF]
        gsem = bufs_sems[NBUF:2 * NBUF]
        wsem = bufs_sems[2 * NBUF:]
        sid = lax.axis_index("s")
        wid = sid * 2 + lax.axis_index("c")
        base = wid * EPW

        rows_per_tile = NPAD // 16
        pltpu.sync_copy(table_hbm.at[pl.ds(sid * rows_per_tile, rows_per_tile)],
                        tab_s.at[pl.ds(sid * rows_per_tile, rows_per_tile)])
        pltpu.sync_copy(idx_hbm.at[pl.ds(base, EPW)], idx_v)
        plsc.subcore_barrier()

        def _drain(sem, b):
            pltpu.make_async_copy(table_hbm.at[pl.ds(0, CH)], rows[b],
                                  sem).wait()

        for b in range(NBUF):
            pltpu.async_copy(tab_s.at[idx_v.at[pl.ds(b * CH, CH)]],
                             rows[b], gsem[b])

        def body(g, carry):
            c0 = g * NBUF
            for b in range(NBUF):
                _drain(gsem[b], b)
                pltpu.async_copy(rows[b],
                                 out_hbm.at[pl.ds(base + (c0 + b) * CH, CH)],
                                 wsem[b])

            @pl.when(g + 1 < NGRP)
            def _():
                for b in range(NBUF):
                    off = (c0 + NBUF + b) * CH
                    _drain(wsem[b], b)
                    pltpu.async_copy(
                        tab_s.at[idx_v.at[pl.ds(off, CH)]],
                        rows[b], gsem[b])

            return carry

        lax.fori_loop(0, NGRP, body, 0)
        for b in range(NBUF):
            _drain(wsem[b], b)

    return k


@functools.lru_cache(maxsize=None)
def _get_gather(ncols):
    return _make_sc_gather(ncols)


def _init_kernel(z_ref, emb_ref, wf_ref, bfv_ref, ws_ref, bsv_ref,
                 x_ref, af_ref, as_ref):
    zc = z_ref[...]
    oh = (zc == lax.broadcasted_iota(jnp.int32, (NPAD, D), 1))
    x = jnp.dot(oh.astype(jnp.float32), emb_ref[...],
                preferred_element_type=jnp.float32)
    x_ref[...] = x
    af_ref[...] = jnp.dot(x, wf_ref[...],
                          preferred_element_type=jnp.float32) + bfv_ref[...]
    as_ref[...] = jnp.dot(x, ws_ref[...],
                          preferred_element_type=jnp.float32) + bsv_ref[...]


def _init_call(zcol, embp, wf0, bf0, ws0, bs0):
    return pl.pallas_call(
        _init_kernel,
        out_shape=[jax.ShapeDtypeStruct((NPAD, D), jnp.float32)] * 3,
    )(zcol, embp, wf0, bf0, ws0, bs0)


def _dist_kernel(posg_ref, posd_ref, dist_ref):
    pdst = jnp.broadcast_to(posd_ref[...][:, None, :],
                            (NB, K, D)).reshape(EB, D)
    diff = posg_ref[...] - pdst
    dist_ref[...] = jnp.sqrt(jnp.sum(diff * diff, axis=1, keepdims=True)
                             + 1e-12)


def _dist_call(posg, pos128):
    return pl.pallas_call(
        _dist_kernel,
        grid=(E // EB,),
        in_specs=[
            pl.BlockSpec((EB, D), lambda i: (i, 0)),
            pl.BlockSpec((NB, D), lambda i: (i, 0)),
        ],
        out_specs=pl.BlockSpec((EB, 1), lambda i: (i, 0)),
        out_shape=jax.ShapeDtypeStruct((E, 1), jnp.float32),
    )(posg, pos128)


def _edge_kernel(xg_ref, dist_ref, af_ref, as_ref, em_ref,
                 wfn_ref, wsn_ref, wfd_ref, wsd_ref, aggr_ref):
    xg = xg_ref[...]
    mmf = jnp.dot(xg, wfn_ref[...], preferred_element_type=jnp.float32)
    mms = jnp.dot(xg, wsn_ref[...], preferred_element_type=jnp.float32)
    dist = dist_ref[...]
    afb = jnp.broadcast_to(af_ref[...][:, None, :], (NB, K, D)).reshape(EB, D)
    asb = jnp.broadcast_to(as_ref[...][:, None, :], (NB, K, D)).reshape(EB, D)
    gate = jax.nn.sigmoid(afb + mmf + dist * wfd_ref[...])
    core = jax.nn.softplus(asb + mms + dist * wsd_ref[...])
    msg = gate * core * em_ref[...]
    aggr_ref[...] = jnp.sum(msg.reshape(NB, K, D), axis=1)


def _edge_call(xg, dist_e, af, asv, em_e, wfn, wsn, wfd, wsd):
    return pl.pallas_call(
        _edge_kernel,
        grid=(E // EB,),
        in_specs=[
            pl.BlockSpec((EB, D), lambda i: (i, 0)),
            pl.BlockSpec((EB, 1), lambda i: (i, 0)),
            pl.BlockSpec((NB, D), lambda i: (i, 0)),
            pl.BlockSpec((NB, D), lambda i: (i, 0)),
            pl.BlockSpec((EB, 1), lambda i: (i, 0)),
            pl.BlockSpec((D, D), lambda i: (0, 0)),
            pl.BlockSpec((D, D), lambda i: (0, 0)),
            pl.BlockSpec((1, D), lambda i: (0, 0)),
            pl.BlockSpec((1, D), lambda i: (0, 0)),
        ],
        out_specs=pl.BlockSpec((NB, D), lambda i: (i, 0)),
        out_shape=jax.ShapeDtypeStruct((NPAD, D), jnp.float32),
    )(xg, dist_e, af, asv, em_e, wfn, wsn, wfd, wsd)


def _bn_kernel(aggr_ref, x_ref, g1_ref, b1_ref, g2_ref, b2_ref,
               wfn_ref, bfn_ref, wsn_ref, bsn_ref,
               xn_ref, af_ref, as_ref):
    rmask = (lax.broadcasted_iota(jnp.int32, (NPAD, 1), 0) < N)
    rmf = rmask.astype(jnp.float32)
    a = aggr_ref[...] * rmf
    s1 = jnp.sum(a, axis=0, keepdims=True) / N
    v1 = jnp.sum(a * a, axis=0, keepdims=True) / N - s1 * s1
    out = (a - s1) / jnp.sqrt(v1 + EPS) * g1_ref[...] + b1_ref[...] \
        + x_ref[...]
    om = out * rmf
    s2 = jnp.sum(om, axis=0, keepdims=True) / N
    v2 = jnp.sum(om * om, axis=0, keepdims=True) / N - s2 * s2
    xn = jnp.maximum((out - s2) / jnp.sqrt(v2 + EPS) * g2_ref[...]
                     + b2_ref[...], 0.0) * rmf
    xn_ref[...] = xn
    af_ref[...] = jnp.dot(xn, wfn_ref[...],
                          preferred_element_type=jnp.float32) + bfn_ref[...]
    as_ref[...] = jnp.dot(xn, wsn_ref[...],
                          preferred_element_type=jnp.float32) + bsn_ref[...]


def _bn_call(aggr, x, g1, b1, g2, b2, wfn, bfn, wsn, bsn):
    return pl.pallas_call(
        _bn_kernel,
        out_shape=[jax.ShapeDtypeStruct((NPAD, D), jnp.float32)] * 3,
    )(aggr, x, g1, b1, g2, b2, wfn, bfn, wsn, bsn)


def _head_kernel(x_ref, bcol_ref, w1_ref, b1_ref, w2_ref, b2_ref, out_ref):
    oh = (bcol_ref[...] == lax.broadcasted_iota(jnp.int32, (NPAD, G), 1))
    ohf = oh.astype(jnp.float32)
    seg = lax.dot_general(ohf, x_ref[...], (((0,), (0,)), ((), ())),
                          preferred_element_type=jnp.float32)
    ones = jnp.ones((NPAD, 1), jnp.float32)
    cnt = lax.dot_general(ohf, ones, (((0,), (0,)), ((), ())),
                          preferred_element_type=jnp.float32)
    gfeat = seg / jnp.maximum(cnt, 1.0)
    h = jnp.maximum(jnp.dot(gfeat, w1_ref[...],
                            preferred_element_type=jnp.float32)
                    + b1_ref[...], 0.0)
    out_ref[...] = jnp.sum(h * w2_ref[...], axis=1, keepdims=True) \
        + b2_ref[...]


def _head_call(x, bcol, hW1, hb1, hw2r, hb2):
    return pl.pallas_call(
        _head_kernel,
        out_shape=jax.ShapeDtypeStruct((G, 1), jnp.float32),
    )(x, bcol, hW1, hb1, hw2r, hb2)


def kernel(z, pos, batch, emb, Wf, bf, Ws, bs, bn1_g, bn1_b, bn2_g, bn2_b,
           hW1, hb1, hW2, hb2):
    batchp = jnp.concatenate(
        [batch.astype(jnp.int32), jnp.full((NPAD - N,), PADB, jnp.int32)])
    posp = jnp.concatenate(
        [pos.astype(jnp.float32), jnp.zeros((NPAD - N, 3), jnp.float32)], 0)
    pos8 = jnp.pad(posp, ((0, 0), (0, 5)))
    posT = pos8.T
    zcol = jnp.concatenate(
        [z.astype(jnp.int32), jnp.zeros((NPAD - N,), jnp.int32)]).reshape(
            NPAD, 1)
    embp = jnp.pad(emb.astype(jnp.float32), ((0, D - emb.shape[0]), (0, 0)))

    bt = batchp.reshape(NRT, RT)
    lo = jnp.searchsorted(batchp, bt.min(axis=1), side="left")
    hi = jnp.searchsorted(batchp, bt.max(axis=1), side="right")
    tlo = (lo // CT).astype(jnp.int32)
    thi = ((hi + CT - 1) // CT).astype(jnp.int32)
    brow = batchp.reshape(NRT, RT, 1)
    batch2d = batchp.reshape(1, NPAD)
    bcol = batchp.reshape(NPAD, 1)

    wf_self = [Wf[l, :D, :] for l in range(L)]
    wf_nbr = [Wf[l, D:2 * D, :] for l in range(L)]
    wf_d = [Wf[l, 2 * D:, :] for l in range(L)]
    ws_self = [Ws[l, :D, :] for l in range(L)]
    ws_nbr = [Ws[l, D:2 * D, :] for l in range(L)]
    ws_d = [Ws[l, 2 * D:, :] for l in range(L)]
    bfv = [bf[l].reshape(1, D) for l in range(L)]
    bsv = [bs[l].reshape(1, D) for l in range(L)]
    g1 = [bn1_g[l].reshape(1, D) for l in range(L)]
    b1 = [bn1_b[l].reshape(1, D) for l in range(L)]
    g2 = [bn2_g[l].reshape(1, D) for l in range(L)]
    b2 = [bn2_b[l].reshape(1, D) for l in range(L)]

    nbr, emask, _ = _build_neighbors(tlo, thi, brow, pos8, posT, batch2d)
    idx_flat = nbr.reshape(E)
    em_e = emask.reshape(E, 1)

    pos128 = jnp.pad(posp, ((0, 0), (0, D - 3)))
    posg = _get_gather(D)(pos128, idx_flat)
    dist_e = _dist_call(posg, pos128)
    x, af, asv = _init_call(zcol, embp, wf_self[0], bfv[0], ws_self[0],
                            bsv[0])
    for l in range(L):
        ln = (l + 1) % L
        xg = _get_gather(D)(x, idx_flat)
        aggr = _edge_call(xg, dist_e, af, asv, em_e,
                          wf_nbr[l], ws_nbr[l], wf_d[l], ws_d[l])
        x, af, asv = _bn_call(aggr, x, g1[l], b1[l], g2[l], b2[l],
                              wf_self[ln], bfv[ln], ws_self[ln], bsv[ln])

    out = _head_call(x, bcol, hW1, hb1.reshape(1, D), hW2.reshape(1, D),
                     hb2.reshape(1, 1))
    return out.reshape(-1)
